# baseline (speedup 1.0000x reference)
import functools

import jax
import jax.numpy as jnp
from jax import lax
from jax.experimental import pallas as pl
from jax.experimental.pallas import tpu as pltpu
from jax.experimental.pallas import tpu_sc as plsc

B, N, K, SCALE = 4, 8192, 32, 4
_EPS = 1e-5
_HIGH = jax.lax.Precision.HIGHEST


def _knn_body(cpad_ref, ppad_ref, cc_ref, pp_ref, idx_ref, *, n, mb):
    b = pl.program_id(0)
    cbf = cpad_ref[0].astype(jnp.bfloat16)
    pbf = ppad_ref[0].astype(jnp.bfloat16)
    cross = jax.lax.dot_general(cbf, pbf, (((1,), (0,)), ((), ())),
                                preferred_element_type=jnp.float32)
    d = (cc_ref[0] + pp_ref[0]) - 2.0 * cross
    iota_n = jax.lax.broadcasted_iota(jnp.int32, (mb, n), 1)
    iota_k = jax.lax.broadcasted_iota(jnp.int32, (mb, K), 1)

    def step(k, carry):
        dcur, knn = carry
        m = jnp.min(dcur, axis=1, keepdims=True)
        am = jnp.min(jnp.where(dcur <= m, iota_n, n), axis=1,
                     keepdims=True)
        knn = jnp.where(iota_k == k, jnp.broadcast_to(am, (mb, K)), knn)
        dcur = jnp.where(iota_n == am, jnp.float32(jnp.inf), dcur)
        return dcur, knn

    _, knn = jax.lax.fori_loop(
        0, K, step, (d, jnp.zeros((mb, K), jnp.int32)))
    idx_ref[0] = knn + b * n


def _knn(cpad, ppad, cc, pp, n, m, mb):
    bb = cpad.shape[0]
    nblk = m // mb
    return pl.pallas_call(
        functools.partial(_knn_body, n=n, mb=mb),
        grid=(bb, nblk),
        in_specs=[
            pl.BlockSpec((1, mb, 8), lambda b, i: (b, i, 0)),
            pl.BlockSpec((1, 8, n), lambda b, i: (b, 0, 0)),
            pl.BlockSpec((1, mb, 1), lambda b, i: (b, i, 0)),
            pl.BlockSpec((1, 1, n), lambda b, i: (b, 0, 0)),
        ],
        out_specs=pl.BlockSpec((1, mb, K), lambda b, i: (b, i, 0)),
        out_shape=jax.ShapeDtypeStruct((bb, m, K), jnp.int32),
        compiler_params=pltpu.CompilerParams(
            dimension_semantics=("arbitrary", "arbitrary")),
    )(cpad, ppad, cc, pp)


def _sc_gather(table, idx_flat, d_width, n_rows):
    info = plsc.get_sparse_core_info()
    nw = info.num_cores * info.num_subcores
    rows_w = n_rows // nw
    chunk = min(rows_w, 73728 // d_width, 128)
    n_sub = rows_w // chunk
    mesh = plsc.VectorSubcoreMesh(core_axis_name="c", subcore_axis_name="s")

    @functools.partial(
        pl.kernel, mesh=mesh,
        out_type=jax.ShapeDtypeStruct((n_rows, d_width), jnp.float32),
        compiler_params=pltpu.CompilerParams(use_tc_tiling_on_sc=False),
        scratch_types=[
            pltpu.VMEM((rows_w,), jnp.int32),
            pltpu.VMEM((chunk, d_width), jnp.float32),
            pltpu.SemaphoreType.DMA,
        ],
    )
    def k(table_hbm, idx_hbm, out_hbm, idx_v, rows_v, sem):
        wid = lax.axis_index("s") * info.num_cores + lax.axis_index("c")
        base = wid * rows_w
        pltpu.sync_copy(idx_hbm.at[pl.ds(base, rows_w)], idx_v)
        for s in range(n_sub):
            off = idx_v if n_sub == 1 else idx_v.at[pl.ds(s * chunk, chunk)]
            pltpu.async_copy(table_hbm.at[off], rows_v, sem).wait()
            pltpu.sync_copy(
                rows_v, out_hbm.at[pl.ds(base + s * chunk, chunk)])

    return k(table, idx_flat)


def _moments_body(g_ref, c_ref, s1_ref, s2_ref, *, cw, mb2, cfeat):
    blk = pl.program_id(1)
    g = g_ref[0]
    c = c_ref[0]
    g3 = g.reshape(mb2, K, cw)
    h3 = g3 - c[:, None, :]
    h = h3.reshape(mb2 * K, cw).astype(jnp.bfloat16)
    s1 = jnp.sum(h.astype(jnp.float32), axis=0, keepdims=True)
    s2 = jax.lax.dot_general(h, h, (((0,), (0,)), ((), ())),
                             preferred_element_type=jnp.float32)

    @pl.when(blk == 0)
    def _():
        s1_ref[0] = s1
        s2_ref[0] = s2

    @pl.when(blk != 0)
    def _():
        s1_ref[0] += s1
        s2_ref[0] += s2


def _conv_a_body(g_ref, c_ref, s1_ref, s2_ref, w0_ref, b0_ref, w1_ref,
                 b1_ref, w2_ref, b2_ref, y2_ref, t1_ref, t2_ref,
                 *, cw, mb2, p_tot, c0, c1, c2):
    blk = pl.program_id(1)
    inv_p = jnp.float32(1.0 / p_tot)
    mu = s1_ref[0] * inv_p
    cov = s2_ref[0] * inv_p - mu.reshape(cw, 1) * mu
    w0b = w0_ref[...].astype(jnp.bfloat16)
    w032 = w0b.astype(jnp.float32)
    x = jax.lax.dot_general(w032, cov, (((1,), (0,)), ((), ())),
                            preferred_element_type=jnp.float32,
                            precision=_HIGH)
    var0 = jnp.sum(x * w032, axis=1, keepdims=True)
    s0 = jax.lax.rsqrt(var0 + _EPS)
    m0 = jax.lax.dot_general(mu, w032, (((1,), (1,)), ((), ())),
                             preferred_element_type=jnp.float32,
                             precision=_HIGH) + b0_ref[...]

    g = g_ref[0]
    c = c_ref[0]
    g3 = g.reshape(mb2, K, cw)
    h = (g3 - c[:, None, :]).reshape(mb2 * K, cw).astype(jnp.bfloat16)
    y0 = jax.lax.dot_general(h, w0b, (((1,), (1,)), ((), ())),
                             preferred_element_type=jnp.float32) + b0_ref[...]
    a0 = jnp.maximum((y0 - m0) * s0.reshape(1, c0), 0.0)
    y1 = jax.lax.dot_general(a0.astype(jnp.bfloat16),
                             w1_ref[...].astype(jnp.bfloat16),
                             (((1,), (1,)), ((), ())),
                             preferred_element_type=jnp.float32
                             ) + b1_ref[...]
    pooled = jnp.max(y1.reshape(mb2, K, c1), axis=1)
    pooled = jnp.broadcast_to(pooled[:, None, :],
                              (mb2, K, c1)).reshape(mb2 * K, c1)
    cat = jnp.concatenate([pooled, y1], axis=1)
    y2 = jax.lax.dot_general(cat.astype(jnp.bfloat16),
                             w2_ref[...].astype(jnp.bfloat16),
                             (((1,), (1,)), ((), ())),
                             preferred_element_type=jnp.float32
                             ) + b2_ref[...]
    y2_ref[0] = y2
    t1 = jnp.sum(y2, axis=0, keepdims=True)
    t2 = jnp.sum(y2 * y2, axis=0, keepdims=True)

    @pl.when(blk == 0)
    def _():
        t1_ref[0] = t1
        t2_ref[0] = t2

    @pl.when(blk != 0)
    def _():
        t1_ref[0] += t1
        t2_ref[0] += t2


def _conv_b_body(y2_ref, t1_ref, t2_ref, w3_ref, b3_ref, y3_ref,
                 u1_ref, u2_ref, *, p_tot, c2, c3):
    blk = pl.program_id(1)
    inv_p = jnp.float32(1.0 / p_tot)
    m2 = t1_ref[0] * inv_p
    v2 = t2_ref[0] * inv_p - m2 * m2
    sc2 = jax.lax.rsqrt(v2 + _EPS)
    a2 = jnp.maximum((y2_ref[0] - m2) * sc2, 0.0)
    y3 = jax.lax.dot_general(a2.astype(jnp.bfloat16),
                             w3_ref[...].astype(jnp.bfloat16),
                             (((1,), (1,)), ((), ())),
                             preferred_element_type=jnp.float32
                             ) + b3_ref[...]
    y3_ref[0] = y3
    u1 = jnp.sum(y3, axis=0, keepdims=True)
    u2 = jnp.sum(y3 * y3, axis=0, keepdims=True)

    @pl.when(blk == 0)
    def _():
        u1_ref[0] = u1
        u2_ref[0] = u2

    @pl.when(blk != 0)
    def _():
        u1_ref[0] += u1
        u2_ref[0] += u2


def _conv_c_body(y3_ref, u1_ref, u2_ref, fo_ref, *, p_tot, mb2, c3):
    inv_p = jnp.float32(1.0 / p_tot)
    m3 = u1_ref[0] * inv_p
    v3 = u2_ref[0] * inv_p - m3 * m3
    sc3 = jax.lax.rsqrt(v3 + _EPS)
    a3 = jnp.maximum((y3_ref[0] - m3) * sc3, 0.0)
    fo_ref[0] = jnp.max(a3.reshape(mb2, K, c3), axis=1)


def _conv_stack(grouped, centers, params, m, mb2, cfeat, cw):
    bb = grouped.shape[0]
    w0, b0, w1, b1, w2, b2, w3, b3 = params
    c0, c1, c2, c3 = w0.shape[0], w1.shape[0], w2.shape[0], w3.shape[0]
    nblk = m // mb2
    pb = mb2 * K
    p_tot = m * K
    dimsem = pltpu.CompilerParams(
        dimension_semantics=("parallel", "arbitrary"))

    s1, s2 = pl.pallas_call(
        functools.partial(_moments_body, cw=cw, mb2=mb2, cfeat=cfeat),
        grid=(bb, nblk),
        in_specs=[
            pl.BlockSpec((1, pb, cw), lambda b, i: (b, i, 0)),
            pl.BlockSpec((1, mb2, cw), lambda b, i: (b, i, 0)),
        ],
        out_specs=[
            pl.BlockSpec((1, 1, cw), lambda b, i: (b, 0, 0)),
            pl.BlockSpec((1, cw, cw), lambda b, i: (b, 0, 0)),
        ],
        out_shape=[
            jax.ShapeDtypeStruct((bb, 1, cw), jnp.float32),
            jax.ShapeDtypeStruct((bb, cw, cw), jnp.float32),
        ],
        compiler_params=dimsem,
    )(grouped, centers)

    full = lambda a: pl.BlockSpec(a.shape, lambda b, i: (0,) * a.ndim)
    y2, t1, t2 = pl.pallas_call(
        functools.partial(_conv_a_body, cw=cw, mb2=mb2, p_tot=p_tot,
                          c0=c0, c1=c1, c2=c2),
        grid=(bb, nblk),
        in_specs=[
            pl.BlockSpec((1, pb, cw), lambda b, i: (b, i, 0)),
            pl.BlockSpec((1, mb2, cw), lambda b, i: (b, i, 0)),
            pl.BlockSpec((1, 1, cw), lambda b, i: (b, 0, 0)),
            pl.BlockSpec((1, cw, cw), lambda b, i: (b, 0, 0)),
            full(w0), full(b0), full(w1), full(b1), full(w2), full(b2),
        ],
        out_specs=[
            pl.BlockSpec((1, pb, c2), lambda b, i: (b, i, 0)),
            pl.BlockSpec((1, 1, c2), lambda b, i: (b, 0, 0)),
            pl.BlockSpec((1, 1, c2), lambda b, i: (b, 0, 0)),
        ],
        out_shape=[
            jax.ShapeDtypeStruct((bb, m * K, c2), jnp.float32),
            jax.ShapeDtypeStruct((bb, 1, c2), jnp.float32),
            jax.ShapeDtypeStruct((bb, 1, c2), jnp.float32),
        ],
        compiler_params=dimsem,
    )(grouped, centers, s1, s2, w0, b0, w1, b1, w2, b2)

    y3, u1, u2 = pl.pallas_call(
        functools.partial(_conv_b_body, p_tot=p_tot, c2=c2, c3=c3),
        grid=(bb, nblk),
        in_specs=[
            pl.BlockSpec((1, pb, c2), lambda b, i: (b, i, 0)),
            pl.BlockSpec((1, 1, c2), lambda b, i: (b, 0, 0)),
            pl.BlockSpec((1, 1, c2), lambda b, i: (b, 0, 0)),
            full(w3), full(b3),
        ],
        out_specs=[
            pl.BlockSpec((1, pb, c3), lambda b, i: (b, i, 0)),
            pl.BlockSpec((1, 1, c3), lambda b, i: (b, 0, 0)),
            pl.BlockSpec((1, 1, c3), lambda b, i: (b, 0, 0)),
        ],
        out_shape=[
            jax.ShapeDtypeStruct((bb, m * K, c3), jnp.float32),
            jax.ShapeDtypeStruct((bb, 1, c3), jnp.float32),
            jax.ShapeDtypeStruct((bb, 1, c3), jnp.float32),
        ],
        compiler_params=dimsem,
    )(y2, t1, t2, w3, b3)

    f_t = pl.pallas_call(
        functools.partial(_conv_c_body, p_tot=p_tot, mb2=mb2, c3=c3),
        grid=(bb, nblk),
        in_specs=[
            pl.BlockSpec((1, pb, c3), lambda b, i: (b, i, 0)),
            pl.BlockSpec((1, 1, c3), lambda b, i: (b, 0, 0)),
            pl.BlockSpec((1, 1, c3), lambda b, i: (b, 0, 0)),
        ],
        out_specs=pl.BlockSpec((1, mb2, c3), lambda b, i: (b, i, 0)),
        out_shape=jax.ShapeDtypeStruct((bb, m, c3), jnp.float32),
        compiler_params=dimsem,
    )(y3, u1, u2)
    return f_t


def _stage(cur_p, cur_ft, m, params_raw, stage_id, mb, mb2):
    bb = cur_p.shape[0]
    n = cur_p.shape[1]
    cfeat = cur_ft.shape[2]
    cw = 8 if cfeat == 3 else 72

    idx = jax.random.randint(jax.random.key(1234 + stage_id), (bb, m), 0, n)
    center_p = jnp.take_along_axis(
        cur_p, jnp.broadcast_to(idx[:, :, None], (bb, m, 3)), axis=1)

    cpad = jnp.concatenate(
        [center_p, jnp.zeros((bb, m, 5), jnp.float32)], axis=2)
    ppad = jnp.concatenate(
        [jnp.swapaxes(cur_p, 1, 2),
         jnp.zeros((bb, 5, n), jnp.float32)], axis=1)
    cc = jnp.sum(center_p ** 2, axis=-1)[:, :, None]
    pp = jnp.sum(cur_p ** 2, axis=-1)[:, None, :]

    knn = _knn(cpad, ppad, cc, pp, n, m, mb)

    table = jnp.concatenate(
        [cur_p, cur_ft,
         jnp.zeros((bb, n, cw - 3 - cfeat), jnp.float32)], axis=2)
    table = table.reshape(bb * n, cw)
    grouped = _sc_gather(table, knn.reshape(bb * m * K), cw,
                         bb * m * K)
    grouped = grouped.reshape(bb, m * K, cw)

    center_ft = jnp.take_along_axis(
        cur_ft, jnp.broadcast_to(idx[:, :, None], (bb, m, cfeat)), axis=1)
    centers = jnp.concatenate(
        [center_p, center_ft,
         jnp.zeros((bb, m, cw - 3 - cfeat), jnp.float32)], axis=2)

    w0, b0, w1, b1, w2, b2, w3, b3 = params_raw
    pad0 = cw - w0.shape[1]
    params = (
        jnp.pad(w0, ((0, 0), (0, pad0))), b0.reshape(1, -1),
        w1, b1.reshape(1, -1), w2, b2.reshape(1, -1),
        w3, b3.reshape(1, -1),
    )
    f_t = _conv_stack(grouped, centers, params, m, mb2, cfeat, cw)
    return center_p, f_t


def kernel(p, f, W00, b00, W01, b01, W02, b02, W03, b03, W10, b10, W11, b11,
           W12, b12, W13, b13):
    m1, m2 = N // SCALE, N // (SCALE * SCALE)
    ft0 = jnp.swapaxes(f, 1, 2)
    p1, f1t = _stage(p, ft0, m1, (W00, b00, W01, b01, W02, b02, W03, b03),
                     0, mb=256, mb2=256)
    p2, f2t = _stage(p1, f1t, m2, (W10, b10, W11, b11, W12, b12, W13, b13),
                     1, mb=256, mb2=128)
    return (p, p1, p2, f, jnp.swapaxes(f1t, 1, 2), jnp.swapaxes(f2t, 1, 2))


# speedup vs baseline: 1.0025x; 1.0025x over previous
import functools

import jax
import jax.numpy as jnp
from jax import lax
from jax.experimental import pallas as pl
from jax.experimental.pallas import tpu as pltpu
from jax.experimental.pallas import tpu_sc as plsc

B, N, K, SCALE = 4, 8192, 32, 4
_EPS = 1e-5
_HIGH = jax.lax.Precision.HIGHEST


def _knn_body(cpad_ref, ppad_ref, cc_ref, pp_ref, idx_ref, *, n, mb):
    b = pl.program_id(0)
    cbf = cpad_ref[0].astype(jnp.bfloat16)
    pbf = ppad_ref[0].astype(jnp.bfloat16)
    cross = jax.lax.dot_general(cbf, pbf, (((1,), (0,)), ((), ())),
                                preferred_element_type=jnp.float32)
    d = (cc_ref[0] + pp_ref[0]) - 2.0 * cross
    iota_n = jax.lax.broadcasted_iota(jnp.int32, (mb, n), 1)
    iota_k = jax.lax.broadcasted_iota(jnp.int32, (mb, K), 1)

    def step(k, carry):
        dcur, knn = carry
        m = jnp.min(dcur, axis=1, keepdims=True)
        am = jnp.min(jnp.where(dcur <= m, iota_n, n), axis=1,
                     keepdims=True)
        knn = jnp.where(iota_k == k, jnp.broadcast_to(am, (mb, K)), knn)
        dcur = jnp.where(iota_n == am, jnp.float32(jnp.inf), dcur)
        return dcur, knn

    _, knn = jax.lax.fori_loop(
        0, K, step, (d, jnp.zeros((mb, K), jnp.int32)))
    idx_ref[0] = knn + b * n


def _knn(cpad, ppad, cc, pp, n, m, mb):
    bb = cpad.shape[0]
    nblk = m // mb
    return pl.pallas_call(
        functools.partial(_knn_body, n=n, mb=mb),
        grid=(bb, nblk),
        in_specs=[
            pl.BlockSpec((1, mb, 8), lambda b, i: (b, i, 0)),
            pl.BlockSpec((1, 8, n), lambda b, i: (b, 0, 0)),
            pl.BlockSpec((1, mb, 1), lambda b, i: (b, i, 0)),
            pl.BlockSpec((1, 1, n), lambda b, i: (b, 0, 0)),
        ],
        out_specs=pl.BlockSpec((1, mb, K), lambda b, i: (b, i, 0)),
        out_shape=jax.ShapeDtypeStruct((bb, m, K), jnp.int32),
        compiler_params=pltpu.CompilerParams(
            dimension_semantics=("parallel", "arbitrary")),
    )(cpad, ppad, cc, pp)


def _sc_gather(table, idx_flat, d_width, n_rows):
    info = plsc.get_sparse_core_info()
    nw = info.num_cores * info.num_subcores
    rows_w = n_rows // nw
    chunk = min(rows_w, 73728 // d_width)
    n_sub = rows_w // chunk
    mesh = plsc.VectorSubcoreMesh(core_axis_name="c", subcore_axis_name="s")

    @functools.partial(
        pl.kernel, mesh=mesh,
        out_type=jax.ShapeDtypeStruct((n_rows, d_width), jnp.float32),
        compiler_params=pltpu.CompilerParams(use_tc_tiling_on_sc=False),
        scratch_types=[
            pltpu.VMEM((rows_w,), jnp.int32),
            pltpu.VMEM((chunk, d_width), jnp.float32),
            pltpu.SemaphoreType.DMA,
        ],
    )
    def k(table_hbm, idx_hbm, out_hbm, idx_v, rows_v, sem):
        wid = lax.axis_index("s") * info.num_cores + lax.axis_index("c")
        base = wid * rows_w
        pltpu.sync_copy(idx_hbm.at[pl.ds(base, rows_w)], idx_v)
        for s in range(n_sub):
            off = idx_v if n_sub == 1 else idx_v.at[pl.ds(s * chunk, chunk)]
            pltpu.async_copy(table_hbm.at[off], rows_v, sem).wait()
            pltpu.sync_copy(
                rows_v, out_hbm.at[pl.ds(base + s * chunk, chunk)])

    return k(table, idx_flat)


def _moments_body(g_ref, c_ref, s1_ref, s2_ref, *, cw, mb2, cfeat):
    blk = pl.program_id(1)
    g = g_ref[0]
    c = c_ref[0]
    g3 = g.reshape(mb2, K, cw)
    h3 = g3 - c[:, None, :]
    h = h3.reshape(mb2 * K, cw).astype(jnp.bfloat16)
    s1 = jnp.sum(h.astype(jnp.float32), axis=0, keepdims=True)
    s2 = jax.lax.dot_general(h, h, (((0,), (0,)), ((), ())),
                             preferred_element_type=jnp.float32)

    @pl.when(blk == 0)
    def _():
        s1_ref[0] = s1
        s2_ref[0] = s2

    @pl.when(blk != 0)
    def _():
        s1_ref[0] += s1
        s2_ref[0] += s2


def _conv_a_body(g_ref, c_ref, s1_ref, s2_ref, w0_ref, b0_ref, w1_ref,
                 b1_ref, w2_ref, b2_ref, y2_ref, t1_ref, t2_ref,
                 *, cw, mb2, p_tot, c0, c1, c2):
    blk = pl.program_id(1)
    inv_p = jnp.float32(1.0 / p_tot)
    mu = s1_ref[0] * inv_p
    cov = s2_ref[0] * inv_p - mu.reshape(cw, 1) * mu
    w0b = w0_ref[...].astype(jnp.bfloat16)
    w032 = w0b.astype(jnp.float32)
    x = jax.lax.dot_general(w032, cov, (((1,), (0,)), ((), ())),
                            preferred_element_type=jnp.float32,
                            precision=_HIGH)
    var0 = jnp.sum(x * w032, axis=1, keepdims=True)
    s0 = jax.lax.rsqrt(var0 + _EPS)
    m0 = jax.lax.dot_general(mu, w032, (((1,), (1,)), ((), ())),
                             preferred_element_type=jnp.float32,
                             precision=_HIGH) + b0_ref[...]

    g = g_ref[0]
    c = c_ref[0]
    g3 = g.reshape(mb2, K, cw)
    h = (g3 - c[:, None, :]).reshape(mb2 * K, cw).astype(jnp.bfloat16)
    y0 = jax.lax.dot_general(h, w0b, (((1,), (1,)), ((), ())),
                             preferred_element_type=jnp.float32) + b0_ref[...]
    a0 = jnp.maximum((y0 - m0) * s0.reshape(1, c0), 0.0)
    y1 = jax.lax.dot_general(a0.astype(jnp.bfloat16),
                             w1_ref[...].astype(jnp.bfloat16),
                             (((1,), (1,)), ((), ())),
                             preferred_element_type=jnp.float32
                             ) + b1_ref[...]
    pooled = jnp.max(y1.reshape(mb2, K, c1), axis=1)
    pooled = jnp.broadcast_to(pooled[:, None, :],
                              (mb2, K, c1)).reshape(mb2 * K, c1)
    cat = jnp.concatenate([pooled, y1], axis=1)
    y2 = jax.lax.dot_general(cat.astype(jnp.bfloat16),
                             w2_ref[...].astype(jnp.bfloat16),
                             (((1,), (1,)), ((), ())),
                             preferred_element_type=jnp.float32
                             ) + b2_ref[...]
    y2_ref[0] = y2
    t1 = jnp.sum(y2, axis=0, keepdims=True)
    t2 = jnp.sum(y2 * y2, axis=0, keepdims=True)

    @pl.when(blk == 0)
    def _():
        t1_ref[0] = t1
        t2_ref[0] = t2

    @pl.when(blk != 0)
    def _():
        t1_ref[0] += t1
        t2_ref[0] += t2


def _conv_b_body(y2_ref, t1_ref, t2_ref, w3_ref, b3_ref, y3_ref,
                 u1_ref, u2_ref, *, p_tot, c2, c3):
    blk = pl.program_id(1)
    inv_p = jnp.float32(1.0 / p_tot)
    m2 = t1_ref[0] * inv_p
    v2 = t2_ref[0] * inv_p - m2 * m2
    sc2 = jax.lax.rsqrt(v2 + _EPS)
    a2 = jnp.maximum((y2_ref[0] - m2) * sc2, 0.0)
    y3 = jax.lax.dot_general(a2.astype(jnp.bfloat16),
                             w3_ref[...].astype(jnp.bfloat16),
                             (((1,), (1,)), ((), ())),
                             preferred_element_type=jnp.float32
                             ) + b3_ref[...]
    y3_ref[0] = y3
    u1 = jnp.sum(y3, axis=0, keepdims=True)
    u2 = jnp.sum(y3 * y3, axis=0, keepdims=True)

    @pl.when(blk == 0)
    def _():
        u1_ref[0] = u1
        u2_ref[0] = u2

    @pl.when(blk != 0)
    def _():
        u1_ref[0] += u1
        u2_ref[0] += u2


def _conv_c_body(y3_ref, u1_ref, u2_ref, fo_ref, *, p_tot, mb2, c3):
    inv_p = jnp.float32(1.0 / p_tot)
    m3 = u1_ref[0] * inv_p
    v3 = u2_ref[0] * inv_p - m3 * m3
    sc3 = jax.lax.rsqrt(v3 + _EPS)
    a3 = jnp.maximum((y3_ref[0] - m3) * sc3, 0.0)
    fo_ref[0] = jnp.max(a3.reshape(mb2, K, c3), axis=1)


def _conv_stack(grouped, centers, params, m, mb2, cfeat, cw):
    bb = grouped.shape[0]
    w0, b0, w1, b1, w2, b2, w3, b3 = params
    c0, c1, c2, c3 = w0.shape[0], w1.shape[0], w2.shape[0], w3.shape[0]
    nblk = m // mb2
    pb = mb2 * K
    p_tot = m * K
    dimsem = pltpu.CompilerParams(
        dimension_semantics=("parallel", "arbitrary"))

    s1, s2 = pl.pallas_call(
        functools.partial(_moments_body, cw=cw, mb2=mb2, cfeat=cfeat),
        grid=(bb, nblk),
        in_specs=[
            pl.BlockSpec((1, pb, cw), lambda b, i: (b, i, 0)),
            pl.BlockSpec((1, mb2, cw), lambda b, i: (b, i, 0)),
        ],
        out_specs=[
            pl.BlockSpec((1, 1, cw), lambda b, i: (b, 0, 0)),
            pl.BlockSpec((1, cw, cw), lambda b, i: (b, 0, 0)),
        ],
        out_shape=[
            jax.ShapeDtypeStruct((bb, 1, cw), jnp.float32),
            jax.ShapeDtypeStruct((bb, cw, cw), jnp.float32),
        ],
        compiler_params=dimsem,
    )(grouped, centers)

    full = lambda a: pl.BlockSpec(a.shape, lambda b, i: (0,) * a.ndim)
    y2, t1, t2 = pl.pallas_call(
        functools.partial(_conv_a_body, cw=cw, mb2=mb2, p_tot=p_tot,
                          c0=c0, c1=c1, c2=c2),
        grid=(bb, nblk),
        in_specs=[
            pl.BlockSpec((1, pb, cw), lambda b, i: (b, i, 0)),
            pl.BlockSpec((1, mb2, cw), lambda b, i: (b, i, 0)),
            pl.BlockSpec((1, 1, cw), lambda b, i: (b, 0, 0)),
            pl.BlockSpec((1, cw, cw), lambda b, i: (b, 0, 0)),
            full(w0), full(b0), full(w1), full(b1), full(w2), full(b2),
        ],
        out_specs=[
            pl.BlockSpec((1, pb, c2), lambda b, i: (b, i, 0)),
            pl.BlockSpec((1, 1, c2), lambda b, i: (b, 0, 0)),
            pl.BlockSpec((1, 1, c2), lambda b, i: (b, 0, 0)),
        ],
        out_shape=[
            jax.ShapeDtypeStruct((bb, m * K, c2), jnp.float32),
            jax.ShapeDtypeStruct((bb, 1, c2), jnp.float32),
            jax.ShapeDtypeStruct((bb, 1, c2), jnp.float32),
        ],
        compiler_params=dimsem,
    )(grouped, centers, s1, s2, w0, b0, w1, b1, w2, b2)

    y3, u1, u2 = pl.pallas_call(
        functools.partial(_conv_b_body, p_tot=p_tot, c2=c2, c3=c3),
        grid=(bb, nblk),
        in_specs=[
            pl.BlockSpec((1, pb, c2), lambda b, i: (b, i, 0)),
            pl.BlockSpec((1, 1, c2), lambda b, i: (b, 0, 0)),
            pl.BlockSpec((1, 1, c2), lambda b, i: (b, 0, 0)),
            full(w3), full(b3),
        ],
        out_specs=[
            pl.BlockSpec((1, pb, c3), lambda b, i: (b, i, 0)),
            pl.BlockSpec((1, 1, c3), lambda b, i: (b, 0, 0)),
            pl.BlockSpec((1, 1, c3), lambda b, i: (b, 0, 0)),
        ],
        out_shape=[
            jax.ShapeDtypeStruct((bb, m * K, c3), jnp.float32),
            jax.ShapeDtypeStruct((bb, 1, c3), jnp.float32),
            jax.ShapeDtypeStruct((bb, 1, c3), jnp.float32),
        ],
        compiler_params=dimsem,
    )(y2, t1, t2, w3, b3)

    f_t = pl.pallas_call(
        functools.partial(_conv_c_body, p_tot=p_tot, mb2=mb2, c3=c3),
        grid=(bb, nblk),
        in_specs=[
            pl.BlockSpec((1, pb, c3), lambda b, i: (b, i, 0)),
            pl.BlockSpec((1, 1, c3), lambda b, i: (b, 0, 0)),
            pl.BlockSpec((1, 1, c3), lambda b, i: (b, 0, 0)),
        ],
        out_specs=pl.BlockSpec((1, mb2, c3), lambda b, i: (b, i, 0)),
        out_shape=jax.ShapeDtypeStruct((bb, m, c3), jnp.float32),
        compiler_params=dimsem,
    )(y3, u1, u2)
    return f_t


def _stage(cur_p, cur_ft, m, params_raw, stage_id, mb, mb2):
    bb = cur_p.shape[0]
    n = cur_p.shape[1]
    cfeat = cur_ft.shape[2]
    cw = 8 if cfeat == 3 else 72

    idx = jax.random.randint(jax.random.key(1234 + stage_id), (bb, m), 0, n)
    center_p = jnp.take_along_axis(
        cur_p, jnp.broadcast_to(idx[:, :, None], (bb, m, 3)), axis=1)

    cpad = jnp.concatenate(
        [center_p, jnp.zeros((bb, m, 5), jnp.float32)], axis=2)
    ppad = jnp.concatenate(
        [jnp.swapaxes(cur_p, 1, 2),
         jnp.zeros((bb, 5, n), jnp.float32)], axis=1)
    cc = jnp.sum(center_p ** 2, axis=-1)[:, :, None]
    pp = jnp.sum(cur_p ** 2, axis=-1)[:, None, :]

    knn = _knn(cpad, ppad, cc, pp, n, m, mb)

    table = jnp.concatenate(
        [cur_p, cur_ft,
         jnp.zeros((bb, n, cw - 3 - cfeat), jnp.float32)], axis=2)
    table = table.reshape(bb * n, cw)
    grouped = _sc_gather(table, knn.reshape(bb * m * K), cw,
                         bb * m * K)
    grouped = grouped.reshape(bb, m * K, cw)

    center_ft = jnp.take_along_axis(
        cur_ft, jnp.broadcast_to(idx[:, :, None], (bb, m, cfeat)), axis=1)
    centers = jnp.concatenate(
        [center_p, center_ft,
         jnp.zeros((bb, m, cw - 3 - cfeat), jnp.float32)], axis=2)

    w0, b0, w1, b1, w2, b2, w3, b3 = params_raw
    pad0 = cw - w0.shape[1]
    params = (
        jnp.pad(w0, ((0, 0), (0, pad0))), b0.reshape(1, -1),
        w1, b1.reshape(1, -1), w2, b2.reshape(1, -1),
        w3, b3.reshape(1, -1),
    )
    f_t = _conv_stack(grouped, centers, params, m, mb2, cfeat, cw)
    return center_p, f_t


def kernel(p, f, W00, b00, W01, b01, W02, b02, W03, b03, W10, b10, W11, b11,
           W12, b12, W13, b13):
    m1, m2 = N // SCALE, N // (SCALE * SCALE)
    ft0 = jnp.swapaxes(f, 1, 2)
    p1, f1t = _stage(p, ft0, m1, (W00, b00, W01, b01, W02, b02, W03, b03),
                     0, mb=256, mb2=256)
    p2, f2t = _stage(p1, f1t, m2, (W10, b10, W11, b11, W12, b12, W13, b13),
                     1, mb=256, mb2=128)
    return (p, p1, p2, f, jnp.swapaxes(f1t, 1, 2), jnp.swapaxes(f2t, 1, 2))


# speedup vs baseline: 1.0242x; 1.0216x over previous
import functools

import jax
import jax.numpy as jnp
from jax import lax
from jax.experimental import pallas as pl
from jax.experimental.pallas import tpu as pltpu
from jax.experimental.pallas import tpu_sc as plsc

B, N, K, SCALE = 4, 8192, 32, 4
_EPS = 1e-5
_HIGH = jax.lax.Precision.HIGHEST


def _knn_body(cpad_ref, ppad_ref, cc_ref, pp_ref, idx_ref, *, n, mb):
    b = pl.program_id(0)
    cbf = cpad_ref[0].astype(jnp.bfloat16)
    pbf = ppad_ref[0].astype(jnp.bfloat16)
    cross = jax.lax.dot_general(cbf, pbf, (((1,), (0,)), ((), ())),
                                preferred_element_type=jnp.float32)
    d = (cc_ref[0] + pp_ref[0]) - 2.0 * cross
    iota_n = jax.lax.broadcasted_iota(jnp.int32, (mb, n), 1)
    iota_k = jax.lax.broadcasted_iota(jnp.int32, (mb, K), 1)

    def step(k, carry):
        dcur, knn = carry
        m = jnp.min(dcur, axis=1, keepdims=True)
        am = jnp.min(jnp.where(dcur <= m, iota_n, n), axis=1,
                     keepdims=True)
        knn = jnp.where(iota_k == k, jnp.broadcast_to(am, (mb, K)), knn)
        dcur = jnp.where(iota_n == am, jnp.float32(jnp.inf), dcur)
        return dcur, knn

    _, knn = jax.lax.fori_loop(
        0, K, step, (d, jnp.zeros((mb, K), jnp.int32)))
    idx_ref[0] = knn + b * n


def _knn(cpad, ppad, cc, pp, n, m, mb):
    bb = cpad.shape[0]
    nblk = m // mb
    return pl.pallas_call(
        functools.partial(_knn_body, n=n, mb=mb),
        grid=(bb, nblk),
        in_specs=[
            pl.BlockSpec((1, mb, 8), lambda b, i: (b, i, 0)),
            pl.BlockSpec((1, 8, n), lambda b, i: (b, 0, 0)),
            pl.BlockSpec((1, mb, 1), lambda b, i: (b, i, 0)),
            pl.BlockSpec((1, 1, n), lambda b, i: (b, 0, 0)),
        ],
        out_specs=pl.BlockSpec((1, mb, K), lambda b, i: (b, i, 0)),
        out_shape=jax.ShapeDtypeStruct((bb, m, K), jnp.int32),
        compiler_params=pltpu.CompilerParams(
            dimension_semantics=("parallel", "arbitrary")),
    )(cpad, ppad, cc, pp)


def _sc_gather(table, idx_flat, d_width, n_rows):
    info = plsc.get_sparse_core_info()
    nw = info.num_cores * info.num_subcores
    rows_w = n_rows // nw
    chunk = min(rows_w, 73728 // d_width)
    n_sub = rows_w // chunk
    mesh = plsc.VectorSubcoreMesh(core_axis_name="c", subcore_axis_name="s")

    @functools.partial(
        pl.kernel, mesh=mesh,
        out_type=jax.ShapeDtypeStruct((n_rows, d_width), jnp.float32),
        compiler_params=pltpu.CompilerParams(use_tc_tiling_on_sc=False),
        scratch_types=[
            pltpu.VMEM((rows_w,), jnp.int32),
            pltpu.VMEM((chunk, d_width), jnp.float32),
            pltpu.SemaphoreType.DMA,
        ],
    )
    def k(table_hbm, idx_hbm, out_hbm, idx_v, rows_v, sem):
        wid = lax.axis_index("s") * info.num_cores + lax.axis_index("c")
        base = wid * rows_w
        pltpu.sync_copy(idx_hbm.at[pl.ds(base, rows_w)], idx_v)
        for s in range(n_sub):
            off = idx_v if n_sub == 1 else idx_v.at[pl.ds(s * chunk, chunk)]
            pltpu.async_copy(table_hbm.at[off], rows_v, sem).wait()
            pltpu.sync_copy(
                rows_v, out_hbm.at[pl.ds(base + s * chunk, chunk)])

    return k(table, idx_flat)


def _moments_body(g_ref, c_ref, s1_ref, s2_ref, *, cw, mb2, cfeat):
    blk = pl.program_id(1)
    g = g_ref[0]
    c = c_ref[0]
    g3 = g.reshape(mb2, K, cw)
    h3 = g3 - c[:, None, :]
    h = h3.reshape(mb2 * K, cw).astype(jnp.bfloat16)
    s1 = jnp.sum(h.astype(jnp.float32), axis=0, keepdims=True)
    s2 = jax.lax.dot_general(h, h, (((0,), (0,)), ((), ())),
                             preferred_element_type=jnp.float32)

    @pl.when(blk == 0)
    def _():
        s1_ref[0] = s1
        s2_ref[0] = s2

    @pl.when(blk != 0)
    def _():
        s1_ref[0] += s1
        s2_ref[0] += s2


def _conv_a_body(g_ref, c_ref, s1_ref, s2_ref, w0_ref, b0_ref, w1_ref,
                 b1_ref, w2_ref, b2_ref, y2_ref, t1_ref, t2_ref,
                 *, cw, mb2, p_tot, c0, c1, c2):
    blk = pl.program_id(1)
    inv_p = jnp.float32(1.0 / p_tot)
    mu = s1_ref[0] * inv_p
    cov = s2_ref[0] * inv_p - mu.reshape(cw, 1) * mu
    w0b = w0_ref[...].astype(jnp.bfloat16)
    w032 = w0b.astype(jnp.float32)
    x = jax.lax.dot_general(w032, cov, (((1,), (0,)), ((), ())),
                            preferred_element_type=jnp.float32,
                            precision=_HIGH)
    var0 = jnp.sum(x * w032, axis=1, keepdims=True)
    s0 = jax.lax.rsqrt(var0 + _EPS)
    m0 = jax.lax.dot_general(mu, w032, (((1,), (1,)), ((), ())),
                             preferred_element_type=jnp.float32,
                             precision=_HIGH) + b0_ref[...]

    g = g_ref[0]
    c = c_ref[0]
    g3 = g.reshape(mb2, K, cw)
    h = (g3 - c[:, None, :]).reshape(mb2 * K, cw).astype(jnp.bfloat16)
    y0 = jax.lax.dot_general(h, w0b, (((1,), (1,)), ((), ())),
                             preferred_element_type=jnp.float32) + b0_ref[...]
    a0 = jnp.maximum((y0 - m0) * s0.reshape(1, c0), 0.0)
    y1 = jax.lax.dot_general(a0.astype(jnp.bfloat16),
                             w1_ref[...].astype(jnp.bfloat16),
                             (((1,), (1,)), ((), ())),
                             preferred_element_type=jnp.float32
                             ) + b1_ref[...]
    pooled = jnp.max(y1.reshape(mb2, K, c1), axis=1)
    pooled = jnp.broadcast_to(pooled[:, None, :],
                              (mb2, K, c1)).reshape(mb2 * K, c1)
    cat = jnp.concatenate([pooled, y1], axis=1)
    y2 = jax.lax.dot_general(cat.astype(jnp.bfloat16),
                             w2_ref[...].astype(jnp.bfloat16),
                             (((1,), (1,)), ((), ())),
                             preferred_element_type=jnp.float32
                             ) + b2_ref[...]
    y2_ref[0] = y2
    t1 = jnp.sum(y2, axis=0, keepdims=True)
    t2 = jnp.sum(y2 * y2, axis=0, keepdims=True)

    @pl.when(blk == 0)
    def _():
        t1_ref[0] = t1
        t2_ref[0] = t2

    @pl.when(blk != 0)
    def _():
        t1_ref[0] += t1
        t2_ref[0] += t2


def _conv_b_body(y2_ref, t1_ref, t2_ref, w3_ref, b3_ref, y3_ref,
                 u1_ref, u2_ref, *, p_tot, c2, c3):
    blk = pl.program_id(1)
    inv_p = jnp.float32(1.0 / p_tot)
    m2 = t1_ref[0] * inv_p
    v2 = t2_ref[0] * inv_p - m2 * m2
    sc2 = jax.lax.rsqrt(v2 + _EPS)
    a2 = jnp.maximum((y2_ref[0] - m2) * sc2, 0.0)
    y3 = jax.lax.dot_general(a2.astype(jnp.bfloat16),
                             w3_ref[...].astype(jnp.bfloat16),
                             (((1,), (1,)), ((), ())),
                             preferred_element_type=jnp.float32
                             ) + b3_ref[...]
    y3_ref[0] = y3
    u1 = jnp.sum(y3, axis=0, keepdims=True)
    u2 = jnp.sum(y3 * y3, axis=0, keepdims=True)

    @pl.when(blk == 0)
    def _():
        u1_ref[0] = u1
        u2_ref[0] = u2

    @pl.when(blk != 0)
    def _():
        u1_ref[0] += u1
        u2_ref[0] += u2


def _conv_c_body(y3_ref, u1_ref, u2_ref, fo_ref, *, p_tot, mb2, c3):
    inv_p = jnp.float32(1.0 / p_tot)
    m3 = u1_ref[0] * inv_p
    v3 = u2_ref[0] * inv_p - m3 * m3
    sc3 = jax.lax.rsqrt(v3 + _EPS)
    a3 = jnp.maximum((y3_ref[0] - m3) * sc3, 0.0)
    fo_ref[0] = jnp.max(a3.reshape(mb2, K, c3), axis=1)


def _conv_stack(grouped, centers, params, m, mb2, cfeat, cw):
    bb = grouped.shape[0]
    w0, b0, w1, b1, w2, b2, w3, b3 = params
    c0, c1, c2, c3 = w0.shape[0], w1.shape[0], w2.shape[0], w3.shape[0]
    nblk = m // mb2
    pb = mb2 * K
    p_tot = m * K
    dimsem = pltpu.CompilerParams(
        dimension_semantics=("parallel", "arbitrary"))

    s1, s2 = pl.pallas_call(
        functools.partial(_moments_body, cw=cw, mb2=mb2, cfeat=cfeat),
        grid=(bb, nblk),
        in_specs=[
            pl.BlockSpec((1, pb, cw), lambda b, i: (b, i, 0)),
            pl.BlockSpec((1, mb2, cw), lambda b, i: (b, i, 0)),
        ],
        out_specs=[
            pl.BlockSpec((1, 1, cw), lambda b, i: (b, 0, 0)),
            pl.BlockSpec((1, cw, cw), lambda b, i: (b, 0, 0)),
        ],
        out_shape=[
            jax.ShapeDtypeStruct((bb, 1, cw), jnp.float32),
            jax.ShapeDtypeStruct((bb, cw, cw), jnp.float32),
        ],
        compiler_params=dimsem,
    )(grouped, centers)

    full = lambda a: pl.BlockSpec(a.shape, lambda b, i: (0,) * a.ndim)
    y2, t1, t2 = pl.pallas_call(
        functools.partial(_conv_a_body, cw=cw, mb2=mb2, p_tot=p_tot,
                          c0=c0, c1=c1, c2=c2),
        grid=(bb, nblk),
        in_specs=[
            pl.BlockSpec((1, pb, cw), lambda b, i: (b, i, 0)),
            pl.BlockSpec((1, mb2, cw), lambda b, i: (b, i, 0)),
            pl.BlockSpec((1, 1, cw), lambda b, i: (b, 0, 0)),
            pl.BlockSpec((1, cw, cw), lambda b, i: (b, 0, 0)),
            full(w0), full(b0), full(w1), full(b1), full(w2), full(b2),
        ],
        out_specs=[
            pl.BlockSpec((1, pb, c2), lambda b, i: (b, i, 0)),
            pl.BlockSpec((1, 1, c2), lambda b, i: (b, 0, 0)),
            pl.BlockSpec((1, 1, c2), lambda b, i: (b, 0, 0)),
        ],
        out_shape=[
            jax.ShapeDtypeStruct((bb, m * K, c2), jnp.float32),
            jax.ShapeDtypeStruct((bb, 1, c2), jnp.float32),
            jax.ShapeDtypeStruct((bb, 1, c2), jnp.float32),
        ],
        compiler_params=dimsem,
    )(grouped, centers, s1, s2, w0, b0, w1, b1, w2, b2)

    y3, u1, u2 = pl.pallas_call(
        functools.partial(_conv_b_body, p_tot=p_tot, c2=c2, c3=c3),
        grid=(bb, nblk),
        in_specs=[
            pl.BlockSpec((1, pb, c2), lambda b, i: (b, i, 0)),
            pl.BlockSpec((1, 1, c2), lambda b, i: (b, 0, 0)),
            pl.BlockSpec((1, 1, c2), lambda b, i: (b, 0, 0)),
            full(w3), full(b3),
        ],
        out_specs=[
            pl.BlockSpec((1, pb, c3), lambda b, i: (b, i, 0)),
            pl.BlockSpec((1, 1, c3), lambda b, i: (b, 0, 0)),
            pl.BlockSpec((1, 1, c3), lambda b, i: (b, 0, 0)),
        ],
        out_shape=[
            jax.ShapeDtypeStruct((bb, m * K, c3), jnp.float32),
            jax.ShapeDtypeStruct((bb, 1, c3), jnp.float32),
            jax.ShapeDtypeStruct((bb, 1, c3), jnp.float32),
        ],
        compiler_params=dimsem,
    )(y2, t1, t2, w3, b3)

    f_t = pl.pallas_call(
        functools.partial(_conv_c_body, p_tot=p_tot, mb2=mb2, c3=c3),
        grid=(bb, nblk),
        in_specs=[
            pl.BlockSpec((1, pb, c3), lambda b, i: (b, i, 0)),
            pl.BlockSpec((1, 1, c3), lambda b, i: (b, 0, 0)),
            pl.BlockSpec((1, 1, c3), lambda b, i: (b, 0, 0)),
        ],
        out_specs=pl.BlockSpec((1, mb2, c3), lambda b, i: (b, i, 0)),
        out_shape=jax.ShapeDtypeStruct((bb, m, c3), jnp.float32),
        compiler_params=dimsem,
    )(y3, u1, u2)
    return f_t


def _stage(cur_p, cur_ft, m, params_raw, stage_id, mb, mb2):
    bb = cur_p.shape[0]
    n = cur_p.shape[1]
    cfeat = cur_ft.shape[2]
    cw = 8 if cfeat == 3 else 72

    idx = jax.random.randint(jax.random.key(1234 + stage_id), (bb, m), 0, n)
    center_p = jnp.take_along_axis(
        cur_p, jnp.broadcast_to(idx[:, :, None], (bb, m, 3)), axis=1)

    cpad = jnp.concatenate(
        [center_p, jnp.zeros((bb, m, 5), jnp.float32)], axis=2)
    ppad = jnp.concatenate(
        [jnp.swapaxes(cur_p, 1, 2),
         jnp.zeros((bb, 5, n), jnp.float32)], axis=1)
    cc = jnp.sum(center_p ** 2, axis=-1)[:, :, None]
    pp = jnp.sum(cur_p ** 2, axis=-1)[:, None, :]

    knn = _knn(cpad, ppad, cc, pp, n, m, mb)

    table = jnp.concatenate(
        [cur_p, cur_ft,
         jnp.zeros((bb, n, cw - 3 - cfeat), jnp.float32)], axis=2)
    table = table.reshape(bb * n, cw)
    grouped = _sc_gather(table, knn.reshape(bb * m * K), cw,
                         bb * m * K)
    grouped = grouped.reshape(bb, m * K, cw)

    center_ft = jnp.take_along_axis(
        cur_ft, jnp.broadcast_to(idx[:, :, None], (bb, m, cfeat)), axis=1)
    centers = jnp.concatenate(
        [center_p, center_ft,
         jnp.zeros((bb, m, cw - 3 - cfeat), jnp.float32)], axis=2)

    w0, b0, w1, b1, w2, b2, w3, b3 = params_raw
    pad0 = cw - w0.shape[1]
    params = (
        jnp.pad(w0, ((0, 0), (0, pad0))), b0.reshape(1, -1),
        w1, b1.reshape(1, -1), w2, b2.reshape(1, -1),
        w3, b3.reshape(1, -1),
    )
    f_t = _conv_stack(grouped, centers, params, m, mb2, cfeat, cw)
    return center_p, f_t


def kernel(p, f, W00, b00, W01, b01, W02, b02, W03, b03, W10, b10, W11, b11,
           W12, b12, W13, b13):
    m1, m2 = N // SCALE, N // (SCALE * SCALE)
    ft0 = jnp.swapaxes(f, 1, 2)
    p1, f1t = _stage(p, ft0, m1, (W00, b00, W01, b01, W02, b02, W03, b03),
                     0, mb=512, mb2=256)
    p2, f2t = _stage(p1, f1t, m2, (W10, b10, W11, b11, W12, b12, W13, b13),
                     1, mb=256, mb2=128)
    return (p, p1, p2, f, jnp.swapaxes(f1t, 1, 2), jnp.swapaxes(f2t, 1, 2))


# speedup vs baseline: 1.0349x; 1.0105x over previous
import functools

import jax
import jax.numpy as jnp
from jax import lax
from jax.experimental import pallas as pl
from jax.experimental.pallas import tpu as pltpu
from jax.experimental.pallas import tpu_sc as plsc

B, N, K, SCALE = 4, 8192, 32, 4
_EPS = 1e-5
_HIGH = jax.lax.Precision.HIGHEST


def _knn_body(cpad_ref, ppad_ref, cc_ref, pp_ref, idx_ref, *, n, mb):
    b = pl.program_id(0)
    cbf = cpad_ref[0].astype(jnp.bfloat16)
    pbf = ppad_ref[0].astype(jnp.bfloat16)
    cross = jax.lax.dot_general(cbf, pbf, (((1,), (0,)), ((), ())),
                                preferred_element_type=jnp.float32)
    d = (cc_ref[0] + pp_ref[0]) - 2.0 * cross
    iota_n = jax.lax.broadcasted_iota(jnp.int32, (mb, n), 1)
    iota_k = jax.lax.broadcasted_iota(jnp.int32, (mb, K), 1)

    def step(k, carry):
        dcur, knn = carry
        am = jnp.argmin(dcur, axis=1).astype(jnp.int32)[:, None]
        knn = jnp.where(iota_k == k, jnp.broadcast_to(am, (mb, K)), knn)
        dcur = jnp.where(iota_n == am, jnp.float32(jnp.inf), dcur)
        return dcur, knn

    _, knn = jax.lax.fori_loop(
        0, K, step, (d, jnp.zeros((mb, K), jnp.int32)))
    idx_ref[0] = knn + b * n


def _knn(cpad, ppad, cc, pp, n, m, mb):
    bb = cpad.shape[0]
    nblk = m // mb
    return pl.pallas_call(
        functools.partial(_knn_body, n=n, mb=mb),
        grid=(bb, nblk),
        in_specs=[
            pl.BlockSpec((1, mb, 8), lambda b, i: (b, i, 0)),
            pl.BlockSpec((1, 8, n), lambda b, i: (b, 0, 0)),
            pl.BlockSpec((1, mb, 1), lambda b, i: (b, i, 0)),
            pl.BlockSpec((1, 1, n), lambda b, i: (b, 0, 0)),
        ],
        out_specs=pl.BlockSpec((1, mb, K), lambda b, i: (b, i, 0)),
        out_shape=jax.ShapeDtypeStruct((bb, m, K), jnp.int32),
        compiler_params=pltpu.CompilerParams(
            dimension_semantics=("parallel", "arbitrary")),
    )(cpad, ppad, cc, pp)


def _sc_gather(table, idx_flat, d_width, n_rows):
    info = plsc.get_sparse_core_info()
    nw = info.num_cores * info.num_subcores
    rows_w = n_rows // nw
    chunk = min(rows_w, 73728 // d_width)
    n_sub = rows_w // chunk
    mesh = plsc.VectorSubcoreMesh(core_axis_name="c", subcore_axis_name="s")

    @functools.partial(
        pl.kernel, mesh=mesh,
        out_type=jax.ShapeDtypeStruct((n_rows, d_width), jnp.float32),
        compiler_params=pltpu.CompilerParams(use_tc_tiling_on_sc=False),
        scratch_types=[
            pltpu.VMEM((rows_w,), jnp.int32),
            pltpu.VMEM((chunk, d_width), jnp.float32),
            pltpu.SemaphoreType.DMA,
        ],
    )
    def k(table_hbm, idx_hbm, out_hbm, idx_v, rows_v, sem):
        wid = lax.axis_index("s") * info.num_cores + lax.axis_index("c")
        base = wid * rows_w
        pltpu.sync_copy(idx_hbm.at[pl.ds(base, rows_w)], idx_v)
        for s in range(n_sub):
            off = idx_v if n_sub == 1 else idx_v.at[pl.ds(s * chunk, chunk)]
            pltpu.async_copy(table_hbm.at[off], rows_v, sem).wait()
            pltpu.sync_copy(
                rows_v, out_hbm.at[pl.ds(base + s * chunk, chunk)])

    return k(table, idx_flat)


def _moments_body(g_ref, c_ref, s1_ref, s2_ref, *, cw, mb2, cfeat):
    blk = pl.program_id(1)
    g = g_ref[0]
    c = c_ref[0]
    g3 = g.reshape(mb2, K, cw)
    h3 = g3 - c[:, None, :]
    h = h3.reshape(mb2 * K, cw).astype(jnp.bfloat16)
    s1 = jnp.sum(h.astype(jnp.float32), axis=0, keepdims=True)
    s2 = jax.lax.dot_general(h, h, (((0,), (0,)), ((), ())),
                             preferred_element_type=jnp.float32)

    @pl.when(blk == 0)
    def _():
        s1_ref[0] = s1
        s2_ref[0] = s2

    @pl.when(blk != 0)
    def _():
        s1_ref[0] += s1
        s2_ref[0] += s2


def _conv_a_body(g_ref, c_ref, s1_ref, s2_ref, w0_ref, b0_ref, w1_ref,
                 b1_ref, w2_ref, b2_ref, y2_ref, t1_ref, t2_ref,
                 *, cw, mb2, p_tot, c0, c1, c2):
    blk = pl.program_id(1)
    inv_p = jnp.float32(1.0 / p_tot)
    mu = s1_ref[0] * inv_p
    cov = s2_ref[0] * inv_p - mu.reshape(cw, 1) * mu
    w0b = w0_ref[...].astype(jnp.bfloat16)
    w032 = w0b.astype(jnp.float32)
    x = jax.lax.dot_general(w032, cov, (((1,), (0,)), ((), ())),
                            preferred_element_type=jnp.float32,
                            precision=_HIGH)
    var0 = jnp.sum(x * w032, axis=1, keepdims=True)
    s0 = jax.lax.rsqrt(var0 + _EPS)
    m0 = jax.lax.dot_general(mu, w032, (((1,), (1,)), ((), ())),
                             preferred_element_type=jnp.float32,
                             precision=_HIGH) + b0_ref[...]

    g = g_ref[0]
    c = c_ref[0]
    g3 = g.reshape(mb2, K, cw)
    h = (g3 - c[:, None, :]).reshape(mb2 * K, cw).astype(jnp.bfloat16)
    y0 = jax.lax.dot_general(h, w0b, (((1,), (1,)), ((), ())),
                             preferred_element_type=jnp.float32) + b0_ref[...]
    a0 = jnp.maximum((y0 - m0) * s0.reshape(1, c0), 0.0)
    y1 = jax.lax.dot_general(a0.astype(jnp.bfloat16),
                             w1_ref[...].astype(jnp.bfloat16),
                             (((1,), (1,)), ((), ())),
                             preferred_element_type=jnp.float32
                             ) + b1_ref[...]
    pooled = jnp.max(y1.reshape(mb2, K, c1), axis=1)
    pooled = jnp.broadcast_to(pooled[:, None, :],
                              (mb2, K, c1)).reshape(mb2 * K, c1)
    cat = jnp.concatenate([pooled, y1], axis=1)
    y2 = jax.lax.dot_general(cat.astype(jnp.bfloat16),
                             w2_ref[...].astype(jnp.bfloat16),
                             (((1,), (1,)), ((), ())),
                             preferred_element_type=jnp.float32
                             ) + b2_ref[...]
    y2_ref[0] = y2
    t1 = jnp.sum(y2, axis=0, keepdims=True)
    t2 = jnp.sum(y2 * y2, axis=0, keepdims=True)

    @pl.when(blk == 0)
    def _():
        t1_ref[0] = t1
        t2_ref[0] = t2

    @pl.when(blk != 0)
    def _():
        t1_ref[0] += t1
        t2_ref[0] += t2


def _conv_b_body(y2_ref, t1_ref, t2_ref, w3_ref, b3_ref, y3_ref,
                 u1_ref, u2_ref, *, p_tot, c2, c3):
    blk = pl.program_id(1)
    inv_p = jnp.float32(1.0 / p_tot)
    m2 = t1_ref[0] * inv_p
    v2 = t2_ref[0] * inv_p - m2 * m2
    sc2 = jax.lax.rsqrt(v2 + _EPS)
    a2 = jnp.maximum((y2_ref[0] - m2) * sc2, 0.0)
    y3 = jax.lax.dot_general(a2.astype(jnp.bfloat16),
                             w3_ref[...].astype(jnp.bfloat16),
                             (((1,), (1,)), ((), ())),
                             preferred_element_type=jnp.float32
                             ) + b3_ref[...]
    y3_ref[0] = y3
    u1 = jnp.sum(y3, axis=0, keepdims=True)
    u2 = jnp.sum(y3 * y3, axis=0, keepdims=True)

    @pl.when(blk == 0)
    def _():
        u1_ref[0] = u1
        u2_ref[0] = u2

    @pl.when(blk != 0)
    def _():
        u1_ref[0] += u1
        u2_ref[0] += u2


def _conv_c_body(y3_ref, u1_ref, u2_ref, fo_ref, *, p_tot, mb2, c3):
    inv_p = jnp.float32(1.0 / p_tot)
    m3 = u1_ref[0] * inv_p
    v3 = u2_ref[0] * inv_p - m3 * m3
    sc3 = jax.lax.rsqrt(v3 + _EPS)
    a3 = jnp.maximum((y3_ref[0] - m3) * sc3, 0.0)
    fo_ref[0] = jnp.max(a3.reshape(mb2, K, c3), axis=1)


def _conv_stack(grouped, centers, params, m, mb2, cfeat, cw):
    bb = grouped.shape[0]
    w0, b0, w1, b1, w2, b2, w3, b3 = params
    c0, c1, c2, c3 = w0.shape[0], w1.shape[0], w2.shape[0], w3.shape[0]
    nblk = m // mb2
    pb = mb2 * K
    p_tot = m * K
    dimsem = pltpu.CompilerParams(
        dimension_semantics=("parallel", "arbitrary"))

    s1, s2 = pl.pallas_call(
        functools.partial(_moments_body, cw=cw, mb2=mb2, cfeat=cfeat),
        grid=(bb, nblk),
        in_specs=[
            pl.BlockSpec((1, pb, cw), lambda b, i: (b, i, 0)),
            pl.BlockSpec((1, mb2, cw), lambda b, i: (b, i, 0)),
        ],
        out_specs=[
            pl.BlockSpec((1, 1, cw), lambda b, i: (b, 0, 0)),
            pl.BlockSpec((1, cw, cw), lambda b, i: (b, 0, 0)),
        ],
        out_shape=[
            jax.ShapeDtypeStruct((bb, 1, cw), jnp.float32),
            jax.ShapeDtypeStruct((bb, cw, cw), jnp.float32),
        ],
        compiler_params=dimsem,
    )(grouped, centers)

    full = lambda a: pl.BlockSpec(a.shape, lambda b, i: (0,) * a.ndim)
    y2, t1, t2 = pl.pallas_call(
        functools.partial(_conv_a_body, cw=cw, mb2=mb2, p_tot=p_tot,
                          c0=c0, c1=c1, c2=c2),
        grid=(bb, nblk),
        in_specs=[
            pl.BlockSpec((1, pb, cw), lambda b, i: (b, i, 0)),
            pl.BlockSpec((1, mb2, cw), lambda b, i: (b, i, 0)),
            pl.BlockSpec((1, 1, cw), lambda b, i: (b, 0, 0)),
            pl.BlockSpec((1, cw, cw), lambda b, i: (b, 0, 0)),
            full(w0), full(b0), full(w1), full(b1), full(w2), full(b2),
        ],
        out_specs=[
            pl.BlockSpec((1, pb, c2), lambda b, i: (b, i, 0)),
            pl.BlockSpec((1, 1, c2), lambda b, i: (b, 0, 0)),
            pl.BlockSpec((1, 1, c2), lambda b, i: (b, 0, 0)),
        ],
        out_shape=[
            jax.ShapeDtypeStruct((bb, m * K, c2), jnp.float32),
            jax.ShapeDtypeStruct((bb, 1, c2), jnp.float32),
            jax.ShapeDtypeStruct((bb, 1, c2), jnp.float32),
        ],
        compiler_params=dimsem,
    )(grouped, centers, s1, s2, w0, b0, w1, b1, w2, b2)

    y3, u1, u2 = pl.pallas_call(
        functools.partial(_conv_b_body, p_tot=p_tot, c2=c2, c3=c3),
        grid=(bb, nblk),
        in_specs=[
            pl.BlockSpec((1, pb, c2), lambda b, i: (b, i, 0)),
            pl.BlockSpec((1, 1, c2), lambda b, i: (b, 0, 0)),
            pl.BlockSpec((1, 1, c2), lambda b, i: (b, 0, 0)),
            full(w3), full(b3),
        ],
        out_specs=[
            pl.BlockSpec((1, pb, c3), lambda b, i: (b, i, 0)),
            pl.BlockSpec((1, 1, c3), lambda b, i: (b, 0, 0)),
            pl.BlockSpec((1, 1, c3), lambda b, i: (b, 0, 0)),
        ],
        out_shape=[
            jax.ShapeDtypeStruct((bb, m * K, c3), jnp.float32),
            jax.ShapeDtypeStruct((bb, 1, c3), jnp.float32),
            jax.ShapeDtypeStruct((bb, 1, c3), jnp.float32),
        ],
        compiler_params=dimsem,
    )(y2, t1, t2, w3, b3)

    f_t = pl.pallas_call(
        functools.partial(_conv_c_body, p_tot=p_tot, mb2=mb2, c3=c3),
        grid=(bb, nblk),
        in_specs=[
            pl.BlockSpec((1, pb, c3), lambda b, i: (b, i, 0)),
            pl.BlockSpec((1, 1, c3), lambda b, i: (b, 0, 0)),
            pl.BlockSpec((1, 1, c3), lambda b, i: (b, 0, 0)),
        ],
        out_specs=pl.BlockSpec((1, mb2, c3), lambda b, i: (b, i, 0)),
        out_shape=jax.ShapeDtypeStruct((bb, m, c3), jnp.float32),
        compiler_params=dimsem,
    )(y3, u1, u2)
    return f_t


def _stage(cur_p, cur_ft, m, params_raw, stage_id, mb, mb2):
    bb = cur_p.shape[0]
    n = cur_p.shape[1]
    cfeat = cur_ft.shape[2]
    cw = 8 if cfeat == 3 else 72

    idx = jax.random.randint(jax.random.key(1234 + stage_id), (bb, m), 0, n)
    center_p = jnp.take_along_axis(
        cur_p, jnp.broadcast_to(idx[:, :, None], (bb, m, 3)), axis=1)

    cpad = jnp.concatenate(
        [center_p, jnp.zeros((bb, m, 5), jnp.float32)], axis=2)
    ppad = jnp.concatenate(
        [jnp.swapaxes(cur_p, 1, 2),
         jnp.zeros((bb, 5, n), jnp.float32)], axis=1)
    cc = jnp.sum(center_p ** 2, axis=-1)[:, :, None]
    pp = jnp.sum(cur_p ** 2, axis=-1)[:, None, :]

    knn = _knn(cpad, ppad, cc, pp, n, m, mb)

    table = jnp.concatenate(
        [cur_p, cur_ft,
         jnp.zeros((bb, n, cw - 3 - cfeat), jnp.float32)], axis=2)
    table = table.reshape(bb * n, cw)
    grouped = _sc_gather(table, knn.reshape(bb * m * K), cw,
                         bb * m * K)
    grouped = grouped.reshape(bb, m * K, cw)

    center_ft = jnp.take_along_axis(
        cur_ft, jnp.broadcast_to(idx[:, :, None], (bb, m, cfeat)), axis=1)
    centers = jnp.concatenate(
        [center_p, center_ft,
         jnp.zeros((bb, m, cw - 3 - cfeat), jnp.float32)], axis=2)

    w0, b0, w1, b1, w2, b2, w3, b3 = params_raw
    pad0 = cw - w0.shape[1]
    params = (
        jnp.pad(w0, ((0, 0), (0, pad0))), b0.reshape(1, -1),
        w1, b1.reshape(1, -1), w2, b2.reshape(1, -1),
        w3, b3.reshape(1, -1),
    )
    f_t = _conv_stack(grouped, centers, params, m, mb2, cfeat, cw)
    return center_p, f_t


def kernel(p, f, W00, b00, W01, b01, W02, b02, W03, b03, W10, b10, W11, b11,
           W12, b12, W13, b13):
    m1, m2 = N // SCALE, N // (SCALE * SCALE)
    ft0 = jnp.swapaxes(f, 1, 2)
    p1, f1t = _stage(p, ft0, m1, (W00, b00, W01, b01, W02, b02, W03, b03),
                     0, mb=512, mb2=256)
    p2, f2t = _stage(p1, f1t, m2, (W10, b10, W11, b11, W12, b12, W13, b13),
                     1, mb=256, mb2=128)
    return (p, p1, p2, f, jnp.swapaxes(f1t, 1, 2), jnp.swapaxes(f2t, 1, 2))


# speedup vs baseline: 1.0443x; 1.0091x over previous
import functools

import jax
import jax.numpy as jnp
from jax import lax
from jax.experimental import pallas as pl
from jax.experimental.pallas import tpu as pltpu
from jax.experimental.pallas import tpu_sc as plsc

B, N, K, SCALE = 4, 8192, 32, 4
_EPS = 1e-5
_HIGH = jax.lax.Precision.HIGHEST


def _knn_body(cpad_ref, ppad_ref, cc_ref, pp_ref, idx_ref, *, n, mb):
    b = pl.program_id(0)
    cbf = cpad_ref[0].astype(jnp.bfloat16)
    pbf = ppad_ref[0].astype(jnp.bfloat16)
    cross = jax.lax.dot_general(cbf, pbf, (((1,), (0,)), ((), ())),
                                preferred_element_type=jnp.float32)
    d = (cc_ref[0] + pp_ref[0]) - 2.0 * cross
    iota_n = jax.lax.broadcasted_iota(jnp.int32, (mb, n), 1)
    iota_k = jax.lax.broadcasted_iota(jnp.int32, (mb, K), 1)

    def step(k, carry):
        dcur, amp, knn = carry
        deff = jnp.where(iota_n == amp, jnp.float32(jnp.inf), dcur)
        am = jnp.argmin(deff, axis=1).astype(jnp.int32)[:, None]
        knn = jnp.where(iota_k == k, jnp.broadcast_to(am, (mb, K)), knn)
        return deff, am, knn

    _, _, knn = jax.lax.fori_loop(
        0, K, step, (d, jnp.full((mb, 1), n, jnp.int32),
                     jnp.zeros((mb, K), jnp.int32)))
    idx_ref[0] = knn + b * n


def _knn(cpad, ppad, cc, pp, n, m, mb):
    bb = cpad.shape[0]
    nblk = m // mb
    return pl.pallas_call(
        functools.partial(_knn_body, n=n, mb=mb),
        grid=(bb, nblk),
        in_specs=[
            pl.BlockSpec((1, mb, 8), lambda b, i: (b, i, 0)),
            pl.BlockSpec((1, 8, n), lambda b, i: (b, 0, 0)),
            pl.BlockSpec((1, mb, 1), lambda b, i: (b, i, 0)),
            pl.BlockSpec((1, 1, n), lambda b, i: (b, 0, 0)),
        ],
        out_specs=pl.BlockSpec((1, mb, K), lambda b, i: (b, i, 0)),
        out_shape=jax.ShapeDtypeStruct((bb, m, K), jnp.int32),
        compiler_params=pltpu.CompilerParams(
            dimension_semantics=("parallel", "arbitrary")),
    )(cpad, ppad, cc, pp)


def _sc_gather(table, idx_flat, d_width, n_rows):
    info = plsc.get_sparse_core_info()
    nw = info.num_cores * info.num_subcores
    rows_w = n_rows // nw
    chunk = min(rows_w, 73728 // d_width)
    n_sub = rows_w // chunk
    mesh = plsc.VectorSubcoreMesh(core_axis_name="c", subcore_axis_name="s")

    @functools.partial(
        pl.kernel, mesh=mesh,
        out_type=jax.ShapeDtypeStruct((n_rows, d_width), jnp.float32),
        compiler_params=pltpu.CompilerParams(use_tc_tiling_on_sc=False),
        scratch_types=[
            pltpu.VMEM((rows_w,), jnp.int32),
            pltpu.VMEM((chunk, d_width), jnp.float32),
            pltpu.SemaphoreType.DMA,
        ],
    )
    def k(table_hbm, idx_hbm, out_hbm, idx_v, rows_v, sem):
        wid = lax.axis_index("s") * info.num_cores + lax.axis_index("c")
        base = wid * rows_w
        pltpu.sync_copy(idx_hbm.at[pl.ds(base, rows_w)], idx_v)
        for s in range(n_sub):
            off = idx_v if n_sub == 1 else idx_v.at[pl.ds(s * chunk, chunk)]
            pltpu.async_copy(table_hbm.at[off], rows_v, sem).wait()
            pltpu.sync_copy(
                rows_v, out_hbm.at[pl.ds(base + s * chunk, chunk)])

    return k(table, idx_flat)


def _moments_body(g_ref, c_ref, s1_ref, s2_ref, *, cw, mb2, cfeat):
    blk = pl.program_id(1)
    g = g_ref[0]
    c = c_ref[0]
    g3 = g.reshape(mb2, K, cw)
    h3 = g3 - c[:, None, :]
    h = h3.reshape(mb2 * K, cw).astype(jnp.bfloat16)
    s1 = jnp.sum(h.astype(jnp.float32), axis=0, keepdims=True)
    s2 = jax.lax.dot_general(h, h, (((0,), (0,)), ((), ())),
                             preferred_element_type=jnp.float32)

    @pl.when(blk == 0)
    def _():
        s1_ref[0] = s1
        s2_ref[0] = s2

    @pl.when(blk != 0)
    def _():
        s1_ref[0] += s1
        s2_ref[0] += s2


def _conv_a_body(g_ref, c_ref, s1_ref, s2_ref, w0_ref, b0_ref, w1_ref,
                 b1_ref, w2_ref, b2_ref, y2_ref, t1_ref, t2_ref,
                 *, cw, mb2, p_tot, c0, c1, c2):
    blk = pl.program_id(1)
    inv_p = jnp.float32(1.0 / p_tot)
    mu = s1_ref[0] * inv_p
    cov = s2_ref[0] * inv_p - mu.reshape(cw, 1) * mu
    w0b = w0_ref[...].astype(jnp.bfloat16)
    w032 = w0b.astype(jnp.float32)
    x = jax.lax.dot_general(w032, cov, (((1,), (0,)), ((), ())),
                            preferred_element_type=jnp.float32,
                            precision=_HIGH)
    var0 = jnp.sum(x * w032, axis=1, keepdims=True)
    s0 = jax.lax.rsqrt(var0 + _EPS)
    m0 = jax.lax.dot_general(mu, w032, (((1,), (1,)), ((), ())),
                             preferred_element_type=jnp.float32,
                             precision=_HIGH) + b0_ref[...]

    g = g_ref[0]
    c = c_ref[0]
    g3 = g.reshape(mb2, K, cw)
    h = (g3 - c[:, None, :]).reshape(mb2 * K, cw).astype(jnp.bfloat16)
    y0 = jax.lax.dot_general(h, w0b, (((1,), (1,)), ((), ())),
                             preferred_element_type=jnp.float32) + b0_ref[...]
    a0 = jnp.maximum((y0 - m0) * s0.reshape(1, c0), 0.0)
    y1 = jax.lax.dot_general(a0.astype(jnp.bfloat16),
                             w1_ref[...].astype(jnp.bfloat16),
                             (((1,), (1,)), ((), ())),
                             preferred_element_type=jnp.float32
                             ) + b1_ref[...]
    pooled = jnp.max(y1.reshape(mb2, K, c1), axis=1)
    pooled = jnp.broadcast_to(pooled[:, None, :],
                              (mb2, K, c1)).reshape(mb2 * K, c1)
    cat = jnp.concatenate([pooled, y1], axis=1)
    y2 = jax.lax.dot_general(cat.astype(jnp.bfloat16),
                             w2_ref[...].astype(jnp.bfloat16),
                             (((1,), (1,)), ((), ())),
                             preferred_element_type=jnp.float32
                             ) + b2_ref[...]
    y2_ref[0] = y2
    t1 = jnp.sum(y2, axis=0, keepdims=True)
    t2 = jnp.sum(y2 * y2, axis=0, keepdims=True)

    @pl.when(blk == 0)
    def _():
        t1_ref[0] = t1
        t2_ref[0] = t2

    @pl.when(blk != 0)
    def _():
        t1_ref[0] += t1
        t2_ref[0] += t2


def _conv_b_body(y2_ref, t1_ref, t2_ref, w3_ref, b3_ref, y3_ref,
                 u1_ref, u2_ref, *, p_tot, c2, c3):
    blk = pl.program_id(1)
    inv_p = jnp.float32(1.0 / p_tot)
    m2 = t1_ref[0] * inv_p
    v2 = t2_ref[0] * inv_p - m2 * m2
    sc2 = jax.lax.rsqrt(v2 + _EPS)
    a2 = jnp.maximum((y2_ref[0] - m2) * sc2, 0.0)
    y3 = jax.lax.dot_general(a2.astype(jnp.bfloat16),
                             w3_ref[...].astype(jnp.bfloat16),
                             (((1,), (1,)), ((), ())),
                             preferred_element_type=jnp.float32
                             ) + b3_ref[...]
    y3_ref[0] = y3
    u1 = jnp.sum(y3, axis=0, keepdims=True)
    u2 = jnp.sum(y3 * y3, axis=0, keepdims=True)

    @pl.when(blk == 0)
    def _():
        u1_ref[0] = u1
        u2_ref[0] = u2

    @pl.when(blk != 0)
    def _():
        u1_ref[0] += u1
        u2_ref[0] += u2


def _conv_c_body(y3_ref, u1_ref, u2_ref, fo_ref, *, p_tot, mb2, c3):
    inv_p = jnp.float32(1.0 / p_tot)
    m3 = u1_ref[0] * inv_p
    v3 = u2_ref[0] * inv_p - m3 * m3
    sc3 = jax.lax.rsqrt(v3 + _EPS)
    a3 = jnp.maximum((y3_ref[0] - m3) * sc3, 0.0)
    fo_ref[0] = jnp.max(a3.reshape(mb2, K, c3), axis=1)


def _conv_stack(grouped, centers, params, m, mb2, cfeat, cw):
    bb = grouped.shape[0]
    w0, b0, w1, b1, w2, b2, w3, b3 = params
    c0, c1, c2, c3 = w0.shape[0], w1.shape[0], w2.shape[0], w3.shape[0]
    nblk = m // mb2
    pb = mb2 * K
    p_tot = m * K
    dimsem = pltpu.CompilerParams(
        dimension_semantics=("parallel", "arbitrary"))

    s1, s2 = pl.pallas_call(
        functools.partial(_moments_body, cw=cw, mb2=mb2, cfeat=cfeat),
        grid=(bb, nblk),
        in_specs=[
            pl.BlockSpec((1, pb, cw), lambda b, i: (b, i, 0)),
            pl.BlockSpec((1, mb2, cw), lambda b, i: (b, i, 0)),
        ],
        out_specs=[
            pl.BlockSpec((1, 1, cw), lambda b, i: (b, 0, 0)),
            pl.BlockSpec((1, cw, cw), lambda b, i: (b, 0, 0)),
        ],
        out_shape=[
            jax.ShapeDtypeStruct((bb, 1, cw), jnp.float32),
            jax.ShapeDtypeStruct((bb, cw, cw), jnp.float32),
        ],
        compiler_params=dimsem,
    )(grouped, centers)

    full = lambda a: pl.BlockSpec(a.shape, lambda b, i: (0,) * a.ndim)
    y2, t1, t2 = pl.pallas_call(
        functools.partial(_conv_a_body, cw=cw, mb2=mb2, p_tot=p_tot,
                          c0=c0, c1=c1, c2=c2),
        grid=(bb, nblk),
        in_specs=[
            pl.BlockSpec((1, pb, cw), lambda b, i: (b, i, 0)),
            pl.BlockSpec((1, mb2, cw), lambda b, i: (b, i, 0)),
            pl.BlockSpec((1, 1, cw), lambda b, i: (b, 0, 0)),
            pl.BlockSpec((1, cw, cw), lambda b, i: (b, 0, 0)),
            full(w0), full(b0), full(w1), full(b1), full(w2), full(b2),
        ],
        out_specs=[
            pl.BlockSpec((1, pb, c2), lambda b, i: (b, i, 0)),
            pl.BlockSpec((1, 1, c2), lambda b, i: (b, 0, 0)),
            pl.BlockSpec((1, 1, c2), lambda b, i: (b, 0, 0)),
        ],
        out_shape=[
            jax.ShapeDtypeStruct((bb, m * K, c2), jnp.float32),
            jax.ShapeDtypeStruct((bb, 1, c2), jnp.float32),
            jax.ShapeDtypeStruct((bb, 1, c2), jnp.float32),
        ],
        compiler_params=dimsem,
    )(grouped, centers, s1, s2, w0, b0, w1, b1, w2, b2)

    y3, u1, u2 = pl.pallas_call(
        functools.partial(_conv_b_body, p_tot=p_tot, c2=c2, c3=c3),
        grid=(bb, nblk),
        in_specs=[
            pl.BlockSpec((1, pb, c2), lambda b, i: (b, i, 0)),
            pl.BlockSpec((1, 1, c2), lambda b, i: (b, 0, 0)),
            pl.BlockSpec((1, 1, c2), lambda b, i: (b, 0, 0)),
            full(w3), full(b3),
        ],
        out_specs=[
            pl.BlockSpec((1, pb, c3), lambda b, i: (b, i, 0)),
            pl.BlockSpec((1, 1, c3), lambda b, i: (b, 0, 0)),
            pl.BlockSpec((1, 1, c3), lambda b, i: (b, 0, 0)),
        ],
        out_shape=[
            jax.ShapeDtypeStruct((bb, m * K, c3), jnp.float32),
            jax.ShapeDtypeStruct((bb, 1, c3), jnp.float32),
            jax.ShapeDtypeStruct((bb, 1, c3), jnp.float32),
        ],
        compiler_params=dimsem,
    )(y2, t1, t2, w3, b3)

    f_t = pl.pallas_call(
        functools.partial(_conv_c_body, p_tot=p_tot, mb2=mb2, c3=c3),
        grid=(bb, nblk),
        in_specs=[
            pl.BlockSpec((1, pb, c3), lambda b, i: (b, i, 0)),
            pl.BlockSpec((1, 1, c3), lambda b, i: (b, 0, 0)),
            pl.BlockSpec((1, 1, c3), lambda b, i: (b, 0, 0)),
        ],
        out_specs=pl.BlockSpec((1, mb2, c3), lambda b, i: (b, i, 0)),
        out_shape=jax.ShapeDtypeStruct((bb, m, c3), jnp.float32),
        compiler_params=dimsem,
    )(y3, u1, u2)
    return f_t


def _stage(cur_p, cur_ft, m, params_raw, stage_id, mb, mb2):
    bb = cur_p.shape[0]
    n = cur_p.shape[1]
    cfeat = cur_ft.shape[2]
    cw = 8 if cfeat == 3 else 72

    idx = jax.random.randint(jax.random.key(1234 + stage_id), (bb, m), 0, n)
    center_p = jnp.take_along_axis(
        cur_p, jnp.broadcast_to(idx[:, :, None], (bb, m, 3)), axis=1)

    cpad = jnp.concatenate(
        [center_p, jnp.zeros((bb, m, 5), jnp.float32)], axis=2)
    ppad = jnp.concatenate(
        [jnp.swapaxes(cur_p, 1, 2),
         jnp.zeros((bb, 5, n), jnp.float32)], axis=1)
    cc = jnp.sum(center_p ** 2, axis=-1)[:, :, None]
    pp = jnp.sum(cur_p ** 2, axis=-1)[:, None, :]

    knn = _knn(cpad, ppad, cc, pp, n, m, mb)

    table = jnp.concatenate(
        [cur_p, cur_ft,
         jnp.zeros((bb, n, cw - 3 - cfeat), jnp.float32)], axis=2)
    table = table.reshape(bb * n, cw)
    grouped = _sc_gather(table, knn.reshape(bb * m * K), cw,
                         bb * m * K)
    grouped = grouped.reshape(bb, m * K, cw)

    center_ft = jnp.take_along_axis(
        cur_ft, jnp.broadcast_to(idx[:, :, None], (bb, m, cfeat)), axis=1)
    centers = jnp.concatenate(
        [center_p, center_ft,
         jnp.zeros((bb, m, cw - 3 - cfeat), jnp.float32)], axis=2)

    w0, b0, w1, b1, w2, b2, w3, b3 = params_raw
    pad0 = cw - w0.shape[1]
    params = (
        jnp.pad(w0, ((0, 0), (0, pad0))), b0.reshape(1, -1),
        w1, b1.reshape(1, -1), w2, b2.reshape(1, -1),
        w3, b3.reshape(1, -1),
    )
    f_t = _conv_stack(grouped, centers, params, m, mb2, cfeat, cw)
    return center_p, f_t


def kernel(p, f, W00, b00, W01, b01, W02, b02, W03, b03, W10, b10, W11, b11,
           W12, b12, W13, b13):
    m1, m2 = N // SCALE, N // (SCALE * SCALE)
    ft0 = jnp.swapaxes(f, 1, 2)
    p1, f1t = _stage(p, ft0, m1, (W00, b00, W01, b01, W02, b02, W03, b03),
                     0, mb=512, mb2=256)
    p2, f2t = _stage(p1, f1t, m2, (W10, b10, W11, b11, W12, b12, W13, b13),
                     1, mb=256, mb2=128)
    return (p, p1, p2, f, jnp.swapaxes(f1t, 1, 2), jnp.swapaxes(f2t, 1, 2))


# speedup vs baseline: 1.0509x; 1.0064x over previous
import functools

import jax
import jax.numpy as jnp
from jax import lax
from jax.experimental import pallas as pl
from jax.experimental.pallas import tpu as pltpu
from jax.experimental.pallas import tpu_sc as plsc

B, N, K, SCALE = 4, 8192, 32, 4
_EPS = 1e-5
_HIGH = jax.lax.Precision.HIGHEST


def _knn_body(cpad_ref, ppad_ref, cc_ref, pp_ref, idx_ref, *, n, mb):
    b = pl.program_id(0)
    cbf = cpad_ref[0].astype(jnp.bfloat16)
    pbf = ppad_ref[0].astype(jnp.bfloat16)
    cross = jax.lax.dot_general(cbf, pbf, (((1,), (0,)), ((), ())),
                                preferred_element_type=jnp.float32)
    d = (cc_ref[0] + pp_ref[0]) - 2.0 * cross
    iota_n = jax.lax.broadcasted_iota(jnp.int32, (mb, n), 1)
    iota_k = jax.lax.broadcasted_iota(jnp.int32, (mb, K), 1)

    def step(k, carry):
        dcur, amp, knn = carry
        deff = jnp.where(iota_n == amp, jnp.float32(jnp.inf), dcur)
        am = jnp.argmin(deff, axis=1).astype(jnp.int32)[:, None]
        knn = jnp.where(iota_k == k, jnp.broadcast_to(am, (mb, K)), knn)
        return deff, am, knn

    _, _, knn = jax.lax.fori_loop(
        0, K, step, (d, jnp.full((mb, 1), n, jnp.int32),
                     jnp.zeros((mb, K), jnp.int32)))
    idx_ref[0] = knn + b * n


def _knn(cpad, ppad, cc, pp, n, m, mb):
    bb = cpad.shape[0]
    nblk = m // mb
    return pl.pallas_call(
        functools.partial(_knn_body, n=n, mb=mb),
        grid=(bb, nblk),
        in_specs=[
            pl.BlockSpec((1, mb, 8), lambda b, i: (b, i, 0)),
            pl.BlockSpec((1, 8, n), lambda b, i: (b, 0, 0)),
            pl.BlockSpec((1, mb, 1), lambda b, i: (b, i, 0)),
            pl.BlockSpec((1, 1, n), lambda b, i: (b, 0, 0)),
        ],
        out_specs=pl.BlockSpec((1, mb, K), lambda b, i: (b, i, 0)),
        out_shape=jax.ShapeDtypeStruct((bb, m, K), jnp.int32),
        compiler_params=pltpu.CompilerParams(
            dimension_semantics=("parallel", "arbitrary")),
    )(cpad, ppad, cc, pp)


def _sc_gather(table, idx_flat, d_width, n_rows):
    info = plsc.get_sparse_core_info()
    nw = info.num_cores * info.num_subcores
    rows_w = n_rows // nw
    chunk = min(rows_w, 73728 // d_width)
    n_sub = rows_w // chunk
    mesh = plsc.VectorSubcoreMesh(core_axis_name="c", subcore_axis_name="s")

    @functools.partial(
        pl.kernel, mesh=mesh,
        out_type=jax.ShapeDtypeStruct((n_rows, d_width), jnp.float32),
        compiler_params=pltpu.CompilerParams(use_tc_tiling_on_sc=False),
        scratch_types=[
            pltpu.VMEM((rows_w,), jnp.int32),
            pltpu.VMEM((chunk, d_width), jnp.float32),
            pltpu.SemaphoreType.DMA,
        ],
    )
    def k(table_hbm, idx_hbm, out_hbm, idx_v, rows_v, sem):
        wid = lax.axis_index("s") * info.num_cores + lax.axis_index("c")
        base = wid * rows_w
        pltpu.sync_copy(idx_hbm.at[pl.ds(base, rows_w)], idx_v)
        for s in range(n_sub):
            off = idx_v if n_sub == 1 else idx_v.at[pl.ds(s * chunk, chunk)]
            pltpu.async_copy(table_hbm.at[off], rows_v, sem).wait()
            pltpu.sync_copy(
                rows_v, out_hbm.at[pl.ds(base + s * chunk, chunk)])

    return k(table, idx_flat)


def _moments_body(g_ref, c_ref, s1_ref, s2_ref, *, cw, mb2, cfeat):
    blk = pl.program_id(1)
    g = g_ref[0]
    c = c_ref[0]
    g3 = g.reshape(mb2, K, cw)
    h3 = g3 - c[:, None, :]
    h = h3.reshape(mb2 * K, cw).astype(jnp.bfloat16)
    s1 = jnp.sum(h.astype(jnp.float32), axis=0, keepdims=True)
    s2 = jax.lax.dot_general(h, h, (((0,), (0,)), ((), ())),
                             preferred_element_type=jnp.float32)

    @pl.when(blk == 0)
    def _():
        s1_ref[0] = s1
        s2_ref[0] = s2

    @pl.when(blk != 0)
    def _():
        s1_ref[0] += s1
        s2_ref[0] += s2


def _conv_a_body(g_ref, c_ref, s1_ref, s2_ref, w0_ref, b0_ref, w1_ref,
                 b1_ref, w2_ref, b2_ref, y2_ref, t1_ref, t2_ref,
                 *, cw, mb2, p_tot, c0, c1, c2):
    blk = pl.program_id(1)
    inv_p = jnp.float32(1.0 / p_tot)
    mu = s1_ref[0] * inv_p
    cov = s2_ref[0] * inv_p - mu.reshape(cw, 1) * mu
    w0b = w0_ref[...].astype(jnp.bfloat16)
    w032 = w0b.astype(jnp.float32)
    x = jax.lax.dot_general(w032, cov, (((1,), (0,)), ((), ())),
                            preferred_element_type=jnp.float32,
                            precision=_HIGH)
    var0 = jnp.sum(x * w032, axis=1, keepdims=True)
    s0 = jax.lax.rsqrt(var0 + _EPS)
    m0 = jax.lax.dot_general(mu, w032, (((1,), (1,)), ((), ())),
                             preferred_element_type=jnp.float32,
                             precision=_HIGH) + b0_ref[...]

    g = g_ref[0]
    c = c_ref[0]
    g3 = g.reshape(mb2, K, cw)
    h = (g3 - c[:, None, :]).reshape(mb2 * K, cw).astype(jnp.bfloat16)
    y0 = jax.lax.dot_general(h, w0b, (((1,), (1,)), ((), ())),
                             preferred_element_type=jnp.float32) + b0_ref[...]
    a0 = jnp.maximum((y0 - m0) * s0.reshape(1, c0), 0.0)
    y1 = jax.lax.dot_general(a0.astype(jnp.bfloat16),
                             w1_ref[...].astype(jnp.bfloat16),
                             (((1,), (1,)), ((), ())),
                             preferred_element_type=jnp.float32
                             ) + b1_ref[...]
    pooled = jnp.max(y1.reshape(mb2, K, c1), axis=1)
    pooled = jnp.broadcast_to(pooled[:, None, :],
                              (mb2, K, c1)).reshape(mb2 * K, c1)
    cat = jnp.concatenate([pooled, y1], axis=1)
    y2 = jax.lax.dot_general(cat.astype(jnp.bfloat16),
                             w2_ref[...].astype(jnp.bfloat16),
                             (((1,), (1,)), ((), ())),
                             preferred_element_type=jnp.float32
                             ) + b2_ref[...]
    y2_ref[0] = y2
    t1 = jnp.sum(y2, axis=0, keepdims=True)
    t2 = jnp.sum(y2 * y2, axis=0, keepdims=True)

    @pl.when(blk == 0)
    def _():
        t1_ref[0] = t1
        t2_ref[0] = t2

    @pl.when(blk != 0)
    def _():
        t1_ref[0] += t1
        t2_ref[0] += t2


def _conv_b_body(y2_ref, t1_ref, t2_ref, w3_ref, b3_ref, y3_ref,
                 u1_ref, u2_ref, *, p_tot, c2, c3):
    blk = pl.program_id(1)
    inv_p = jnp.float32(1.0 / p_tot)
    m2 = t1_ref[0] * inv_p
    v2 = t2_ref[0] * inv_p - m2 * m2
    sc2 = jax.lax.rsqrt(v2 + _EPS)
    a2 = jnp.maximum((y2_ref[0] - m2) * sc2, 0.0)
    y3 = jax.lax.dot_general(a2.astype(jnp.bfloat16),
                             w3_ref[...].astype(jnp.bfloat16),
                             (((1,), (1,)), ((), ())),
                             preferred_element_type=jnp.float32
                             ) + b3_ref[...]
    y3_ref[0] = y3
    u1 = jnp.sum(y3, axis=0, keepdims=True)
    u2 = jnp.sum(y3 * y3, axis=0, keepdims=True)

    @pl.when(blk == 0)
    def _():
        u1_ref[0] = u1
        u2_ref[0] = u2

    @pl.when(blk != 0)
    def _():
        u1_ref[0] += u1
        u2_ref[0] += u2


def _conv_c_body(y3_ref, u1_ref, u2_ref, fo_ref, *, p_tot, mb2, c3):
    inv_p = jnp.float32(1.0 / p_tot)
    m3 = u1_ref[0] * inv_p
    v3 = u2_ref[0] * inv_p - m3 * m3
    sc3 = jax.lax.rsqrt(v3 + _EPS)
    a3 = jnp.maximum((y3_ref[0] - m3) * sc3, 0.0)
    fo_ref[0] = jnp.max(a3.reshape(mb2, K, c3), axis=1)


def _conv_stack(grouped, centers, params, m, mb2, cfeat, cw):
    bb = grouped.shape[0]
    w0, b0, w1, b1, w2, b2, w3, b3 = params
    c0, c1, c2, c3 = w0.shape[0], w1.shape[0], w2.shape[0], w3.shape[0]
    nblk = m // mb2
    pb = mb2 * K
    p_tot = m * K
    dimsem = pltpu.CompilerParams(
        dimension_semantics=("parallel", "arbitrary"))

    s1, s2 = pl.pallas_call(
        functools.partial(_moments_body, cw=cw, mb2=mb2, cfeat=cfeat),
        grid=(bb, nblk),
        in_specs=[
            pl.BlockSpec((1, pb, cw), lambda b, i: (b, i, 0)),
            pl.BlockSpec((1, mb2, cw), lambda b, i: (b, i, 0)),
        ],
        out_specs=[
            pl.BlockSpec((1, 1, cw), lambda b, i: (b, 0, 0)),
            pl.BlockSpec((1, cw, cw), lambda b, i: (b, 0, 0)),
        ],
        out_shape=[
            jax.ShapeDtypeStruct((bb, 1, cw), jnp.float32),
            jax.ShapeDtypeStruct((bb, cw, cw), jnp.float32),
        ],
        compiler_params=dimsem,
    )(grouped, centers)

    full = lambda a: pl.BlockSpec(a.shape, lambda b, i: (0,) * a.ndim)
    y2, t1, t2 = pl.pallas_call(
        functools.partial(_conv_a_body, cw=cw, mb2=mb2, p_tot=p_tot,
                          c0=c0, c1=c1, c2=c2),
        grid=(bb, nblk),
        in_specs=[
            pl.BlockSpec((1, pb, cw), lambda b, i: (b, i, 0)),
            pl.BlockSpec((1, mb2, cw), lambda b, i: (b, i, 0)),
            pl.BlockSpec((1, 1, cw), lambda b, i: (b, 0, 0)),
            pl.BlockSpec((1, cw, cw), lambda b, i: (b, 0, 0)),
            full(w0), full(b0), full(w1), full(b1), full(w2), full(b2),
        ],
        out_specs=[
            pl.BlockSpec((1, pb, c2), lambda b, i: (b, i, 0)),
            pl.BlockSpec((1, 1, c2), lambda b, i: (b, 0, 0)),
            pl.BlockSpec((1, 1, c2), lambda b, i: (b, 0, 0)),
        ],
        out_shape=[
            jax.ShapeDtypeStruct((bb, m * K, c2), jnp.float32),
            jax.ShapeDtypeStruct((bb, 1, c2), jnp.float32),
            jax.ShapeDtypeStruct((bb, 1, c2), jnp.float32),
        ],
        compiler_params=dimsem,
    )(grouped, centers, s1, s2, w0, b0, w1, b1, w2, b2)

    y3, u1, u2 = pl.pallas_call(
        functools.partial(_conv_b_body, p_tot=p_tot, c2=c2, c3=c3),
        grid=(bb, nblk),
        in_specs=[
            pl.BlockSpec((1, pb, c2), lambda b, i: (b, i, 0)),
            pl.BlockSpec((1, 1, c2), lambda b, i: (b, 0, 0)),
            pl.BlockSpec((1, 1, c2), lambda b, i: (b, 0, 0)),
            full(w3), full(b3),
        ],
        out_specs=[
            pl.BlockSpec((1, pb, c3), lambda b, i: (b, i, 0)),
            pl.BlockSpec((1, 1, c3), lambda b, i: (b, 0, 0)),
            pl.BlockSpec((1, 1, c3), lambda b, i: (b, 0, 0)),
        ],
        out_shape=[
            jax.ShapeDtypeStruct((bb, m * K, c3), jnp.float32),
            jax.ShapeDtypeStruct((bb, 1, c3), jnp.float32),
            jax.ShapeDtypeStruct((bb, 1, c3), jnp.float32),
        ],
        compiler_params=dimsem,
    )(y2, t1, t2, w3, b3)

    f_t = pl.pallas_call(
        functools.partial(_conv_c_body, p_tot=p_tot, mb2=mb2, c3=c3),
        grid=(bb, nblk),
        in_specs=[
            pl.BlockSpec((1, pb, c3), lambda b, i: (b, i, 0)),
            pl.BlockSpec((1, 1, c3), lambda b, i: (b, 0, 0)),
            pl.BlockSpec((1, 1, c3), lambda b, i: (b, 0, 0)),
        ],
        out_specs=pl.BlockSpec((1, mb2, c3), lambda b, i: (b, i, 0)),
        out_shape=jax.ShapeDtypeStruct((bb, m, c3), jnp.float32),
        compiler_params=dimsem,
    )(y3, u1, u2)
    return f_t


def _stage(cur_p, cur_ft, m, params_raw, stage_id, mb, mb2):
    bb = cur_p.shape[0]
    n = cur_p.shape[1]
    cfeat = cur_ft.shape[2]
    cw = 8 if cfeat == 3 else 72

    idx = jax.random.randint(jax.random.key(1234 + stage_id), (bb, m), 0, n)
    center_p = jnp.take_along_axis(
        cur_p, jnp.broadcast_to(idx[:, :, None], (bb, m, 3)), axis=1)

    cpad = jnp.concatenate(
        [center_p, jnp.zeros((bb, m, 5), jnp.float32)], axis=2)
    ppad = jnp.concatenate(
        [jnp.swapaxes(cur_p, 1, 2),
         jnp.zeros((bb, 5, n), jnp.float32)], axis=1)
    cc = jnp.sum(center_p ** 2, axis=-1)[:, :, None]
    pp = jnp.sum(cur_p ** 2, axis=-1)[:, None, :]

    knn = _knn(cpad, ppad, cc, pp, n, m, mb)

    table = jnp.concatenate(
        [cur_p, cur_ft,
         jnp.zeros((bb, n, cw - 3 - cfeat), jnp.float32)], axis=2)
    table = table.reshape(bb * n, cw)
    grouped = _sc_gather(table, knn.reshape(bb * m * K), cw,
                         bb * m * K)
    grouped = grouped.reshape(bb, m * K, cw)

    center_ft = jnp.take_along_axis(
        cur_ft, jnp.broadcast_to(idx[:, :, None], (bb, m, cfeat)), axis=1)
    centers = jnp.concatenate(
        [center_p, center_ft,
         jnp.zeros((bb, m, cw - 3 - cfeat), jnp.float32)], axis=2)

    w0, b0, w1, b1, w2, b2, w3, b3 = params_raw
    pad0 = cw - w0.shape[1]
    params = (
        jnp.pad(w0, ((0, 0), (0, pad0))), b0.reshape(1, -1),
        w1, b1.reshape(1, -1), w2, b2.reshape(1, -1),
        w3, b3.reshape(1, -1),
    )
    f_t = _conv_stack(grouped, centers, params, m, mb2, cfeat, cw)
    return center_p, f_t


def kernel(p, f, W00, b00, W01, b01, W02, b02, W03, b03, W10, b10, W11, b11,
           W12, b12, W13, b13):
    m1, m2 = N // SCALE, N // (SCALE * SCALE)
    ft0 = jnp.swapaxes(f, 1, 2)
    p1, f1t = _stage(p, ft0, m1, (W00, b00, W01, b01, W02, b02, W03, b03),
                     0, mb=512, mb2=256)
    p2, f2t = _stage(p1, f1t, m2, (W10, b10, W11, b11, W12, b12, W13, b13),
                     1, mb=512, mb2=128)
    return (p, p1, p2, f, jnp.swapaxes(f1t, 1, 2), jnp.swapaxes(f2t, 1, 2))
